# baseline (device time: 96150 ns/iter reference)
import jax
import jax.numpy as jnp
from jax import lax
from jax.experimental import pallas as pl
from jax.experimental.pallas import tpu as pltpu

N_DEV = 32
PLANE = 8
COL = 4

CYC = [0, 1, 2, 5, 6, 7, 4, 3]
POS = [0, 1, 2, 7, 6, 3, 4, 5]

WA = 640
WB = 384
A_LO, A_HI = 0, WA
B_LO, B_HI = 2 * WA, 2 * WA + WB


def kernel(x, w_mat, scale_x, scale_w):
    m_glob, k_loc = x.shape
    _, n = w_mat.shape
    m_blk = m_glob // N_DEV

    d = lax.axis_index("i")
    z = d // PLANE
    idx = d % PLANE
    cyc = jnp.array(CYC, jnp.int32)
    q = jnp.array(POS, jnp.int32)[idx]
    succ = z * PLANE + cyc[(q + 1) % PLANE]
    pred = z * PLANE + cyc[(q + 7) % PLANE]
    zsucc = ((z + 1) % COL) * PLANE + idx
    zpred = ((z + 3) % COL) * PLANE + idx
    meta = jnp.concatenate(
        [jnp.stack([z, q, succ, pred, zsucc, zpred]).astype(jnp.int32), cyc]
    )

    def body(x_ref, w_ref, sx_ref, sw_ref, meta_ref, out_ref,
             ga_p, ga_m, gb_p, gb_m, ha_p, ha_m, hb_p, hb_m,
             a1sp, a1rp, a1sm, a1rm, a2sp, a2rp, a2sm, a2rm,
             b1sp, b1rp, b1sm, b1rm, b2sp, b2rp, b2sm, b2rm):
        z = meta_ref[0]
        q = meta_ref[1]
        succ = meta_ref[2]
        pred = meta_ref[3]
        zsucc = meta_ref[4]
        zpred = meta_ref[5]

        barrier = pltpu.get_barrier_semaphore()
        for nbr in (succ, pred, zsucc, zpred):
            pl.semaphore_signal(barrier, inc=1, device_id=(nbr,),
                                device_id_type=pl.DeviceIdType.MESH)
        pl.semaphore_wait(barrier, 4)

        def pmat(c, off, wdt):
            xb = x_ref[pl.ds(c * m_blk, m_blk), :]
            return lax.dot_general(
                xb, w_ref[:, off:off + wdt], (((1,), (0,)), ((), ())),
                preferred_element_type=jnp.float32)

        def cyc_at(p):
            return meta_ref[6 + p]

        def rdma(buf, sends, recvs, slot_s, slot_d, lane, tgt):
            if lane is None:
                return pltpu.make_async_remote_copy(
                    src_ref=buf.at[slot_s], dst_ref=buf.at[slot_d],
                    send_sem=sends.at[slot_s], recv_sem=recvs.at[slot_s],
                    device_id=(tgt,), device_id_type=pl.DeviceIdType.MESH)
            return pltpu.make_async_remote_copy(
                src_ref=buf.at[slot_s, lane], dst_ref=buf.at[slot_d, lane],
                send_sem=sends.at[slot_s, lane], recv_sem=recvs.at[slot_s, lane],
                device_id=(tgt,), device_id_type=pl.DeviceIdType.MESH)

        bf = jnp.bfloat16
        f32 = jnp.float32

        iap = cyc_at(lax.rem(q + 7, PLANE))
        iam = cyc_at(lax.rem(q + 1, PLANE))
        for zc in range(COL):
            ga_p[0, zc] = pmat(zc * PLANE + iap, A_LO, WA).astype(bf)
            rdma(ga_p, a1sp, a1rp, 0, 1, zc, succ).start()
            ga_m[0, zc] = pmat(zc * PLANE + iam, A_HI, WA).astype(bf)
            rdma(ga_m, a1sm, a1rm, 0, 1, zc, pred).start()

        zbp = lax.rem(z + 3, COL)
        zbm = lax.rem(z + 1, COL)
        for i in range(PLANE):
            gb_p[0, i] = pmat(zbp * PLANE + i, B_LO, WB).astype(bf)
            rdma(gb_p, b1sp, b1rp, 0, 1, i, zsucc).start()
            gb_m[0, i] = pmat(zbm * PLANE + i, B_HI, WB).astype(bf)
            rdma(gb_m, b1sm, b1rm, 0, 1, i, zpred).start()

        def a1_hop(h, fire_a2=False):
            ip = cyc_at(lax.rem(q + 15 - h, PLANE))
            im = cyc_at(lax.rem(q + h + 1, PLANE))
            nxt = (pmat(ip, A_LO, WA), pmat(im, A_HI, WA))
            for zc in range(COL):
                pp, pm = nxt
                if zc < COL - 1:
                    nxt = (pmat((zc + 1) * PLANE + ip, A_LO, WA),
                           pmat((zc + 1) * PLANE + im, A_HI, WA))
                rdma(ga_p, a1sp, a1rp, h - 1, h, zc, succ).wait_recv()
                ga_p[h, zc] = (ga_p[h, zc].astype(f32) + pp).astype(bf)
                if h < PLANE - 1:
                    rdma(ga_p, a1sp, a1rp, h, h + 1, zc, succ).start()
                rdma(ga_m, a1sm, a1rm, h - 1, h, zc, pred).wait_recv()
                ga_m[h, zc] = (ga_m[h, zc].astype(f32) + pm).astype(bf)
                if h < PLANE - 1:
                    rdma(ga_m, a1sm, a1rm, h, h + 1, zc, pred).start()
                if fire_a2:
                    @pl.when(zc == lax.rem(z + 3, COL))
                    def _():
                        ha_p[0] = ga_p[PLANE - 1, zc]
                        rdma(ha_p, a2sp, a2rp, 0, 1, None, zsucc).start()

                    @pl.when(zc == lax.rem(z + 1, COL))
                    def _():
                        ha_m[0] = ga_m[PLANE - 1, zc]
                        rdma(ha_m, a2sm, a2rm, 0, 1, None, zpred).start()

        def b1_hop(t, fire_b2=False):
            zp = lax.rem(z + 7 - t, COL)
            zm = lax.rem(z + t + 1, COL)
            nxt = (pmat(zp * PLANE, B_LO, WB), pmat(zm * PLANE, B_HI, WB))
            for i in range(PLANE):
                pp, pm = nxt
                if i < PLANE - 1:
                    nxt = (pmat(zp * PLANE + i + 1, B_LO, WB),
                           pmat(zm * PLANE + i + 1, B_HI, WB))
                rdma(gb_p, b1sp, b1rp, t - 1, t, i, zsucc).wait_recv()
                gb_p[t, i] = (gb_p[t, i].astype(f32) + pp).astype(bf)
                if t < COL - 1:
                    rdma(gb_p, b1sp, b1rp, t, t + 1, i, zsucc).start()
                rdma(gb_m, b1sm, b1rm, t - 1, t, i, zpred).wait_recv()
                gb_m[t, i] = (gb_m[t, i].astype(f32) + pm).astype(bf)
                if t < COL - 1:
                    rdma(gb_m, b1sm, b1rm, t, t + 1, i, zpred).start()
                if fire_b2:
                    @pl.when(i == cyc_at(lax.rem(q + 7, PLANE)))
                    def _():
                        hb_p[0] = gb_p[COL - 1, i]
                        rdma(hb_p, b2sp, b2rp, 0, 1, None, succ).start()

                    @pl.when(i == cyc_at(lax.rem(q + 1, PLANE)))
                    def _():
                        hb_m[0] = gb_m[COL - 1, i]
                        rdma(hb_m, b2sm, b2rm, 0, 1, None, pred).start()

        a1_hop(1)
        a1_hop(2)
        b1_hop(1)
        a1_hop(3)
        a1_hop(4)
        b1_hop(2)
        a1_hop(5)
        a1_hop(6)
        b1_hop(3, fire_b2=True)
        a1_hop(7, fire_a2=True)

        s = sx_ref[0] * sw_ref[0]

        def b2_hop(u):
            addp = gb_p[COL - 1, cyc_at(lax.rem(q + 15 - u, PLANE))].astype(f32)
            addm = gb_m[COL - 1, cyc_at(lax.rem(q + u + 1, PLANE))].astype(f32)
            rdma(hb_p, b2sp, b2rp, u - 1, u, None, succ).wait_recv()
            accp = hb_p[u].astype(f32) + addp
            if u < PLANE - 1:
                hb_p[u] = accp.astype(bf)
                rdma(hb_p, b2sp, b2rp, u, u + 1, None, succ).start()
            else:
                out_ref[:, B_LO:B_LO + WB] = jnp.maximum(accp * s, 0.0)
            rdma(hb_m, b2sm, b2rm, u - 1, u, None, pred).wait_recv()
            accm = hb_m[u].astype(f32) + addm
            if u < PLANE - 1:
                hb_m[u] = accm.astype(bf)
                rdma(hb_m, b2sm, b2rm, u, u + 1, None, pred).start()
            else:
                out_ref[:, B_HI:B_HI + WB] = jnp.maximum(accm * s, 0.0)

        def a2_hop(t):
            addp = ga_p[PLANE - 1, lax.rem(z + 7 - t, COL)].astype(f32)
            addm = ga_m[PLANE - 1, lax.rem(z + t + 1, COL)].astype(f32)
            rdma(ha_p, a2sp, a2rp, t - 1, t, None, zsucc).wait_recv()
            accp = ha_p[t].astype(f32) + addp
            if t < COL - 1:
                ha_p[t] = accp.astype(bf)
                rdma(ha_p, a2sp, a2rp, t, t + 1, None, zsucc).start()
            else:
                out_ref[:, A_LO:A_LO + WA] = jnp.maximum(accp * s, 0.0)
            rdma(ha_m, a2sm, a2rm, t - 1, t, None, zpred).wait_recv()
            accm = ha_m[t].astype(f32) + addm
            if t < COL - 1:
                ha_m[t] = accm.astype(bf)
                rdma(ha_m, a2sm, a2rm, t, t + 1, None, zpred).start()
            else:
                out_ref[:, A_HI:A_HI + WA] = jnp.maximum(accm * s, 0.0)

        b2_hop(1)
        a2_hop(1)
        b2_hop(2)
        a2_hop(2)
        b2_hop(3)
        a2_hop(3)
        b2_hop(4)
        b2_hop(5)
        b2_hop(6)
        b2_hop(7)

        for h in range(1, PLANE):
            for zc in range(COL):
                rdma(ga_p, a1sp, a1rp, h - 1, h, zc, succ).wait_send()
                rdma(ga_m, a1sm, a1rm, h - 1, h, zc, pred).wait_send()
        for t in range(1, COL):
            for i in range(PLANE):
                rdma(gb_p, b1sp, b1rp, t - 1, t, i, zsucc).wait_send()
                rdma(gb_m, b1sm, b1rm, t - 1, t, i, zpred).wait_send()
        for u in range(1, PLANE):
            rdma(hb_p, b2sp, b2rp, u - 1, u, None, succ).wait_send()
            rdma(hb_m, b2sm, b2rm, u - 1, u, None, pred).wait_send()
        for t in range(1, COL):
            rdma(ha_p, a2sp, a2rp, t - 1, t, None, zsucc).wait_send()
            rdma(ha_m, a2sm, a2rm, t - 1, t, None, zpred).wait_send()

    return pl.pallas_call(
        body,
        out_shape=jax.ShapeDtypeStruct((m_blk, n), jnp.float32),
        in_specs=[
            pl.BlockSpec(memory_space=pltpu.VMEM),
            pl.BlockSpec(memory_space=pltpu.VMEM),
            pl.BlockSpec(memory_space=pltpu.SMEM),
            pl.BlockSpec(memory_space=pltpu.SMEM),
            pl.BlockSpec(memory_space=pltpu.SMEM),
        ],
        out_specs=pl.BlockSpec(memory_space=pltpu.VMEM),
        scratch_shapes=[
            pltpu.VMEM((PLANE, COL, m_blk, WA), jnp.bfloat16),
            pltpu.VMEM((PLANE, COL, m_blk, WA), jnp.bfloat16),
            pltpu.VMEM((COL, PLANE, m_blk, WB), jnp.bfloat16),
            pltpu.VMEM((COL, PLANE, m_blk, WB), jnp.bfloat16),
            pltpu.VMEM((COL, m_blk, WA), jnp.bfloat16),
            pltpu.VMEM((COL, m_blk, WA), jnp.bfloat16),
            pltpu.VMEM((PLANE, m_blk, WB), jnp.bfloat16),
            pltpu.VMEM((PLANE, m_blk, WB), jnp.bfloat16),
            pltpu.SemaphoreType.DMA((PLANE - 1, COL)),
            pltpu.SemaphoreType.DMA((PLANE - 1, COL)),
            pltpu.SemaphoreType.DMA((PLANE - 1, COL)),
            pltpu.SemaphoreType.DMA((PLANE - 1, COL)),
            pltpu.SemaphoreType.DMA((COL - 1,)),
            pltpu.SemaphoreType.DMA((COL - 1,)),
            pltpu.SemaphoreType.DMA((COL - 1,)),
            pltpu.SemaphoreType.DMA((COL - 1,)),
            pltpu.SemaphoreType.DMA((COL - 1, PLANE)),
            pltpu.SemaphoreType.DMA((COL - 1, PLANE)),
            pltpu.SemaphoreType.DMA((COL - 1, PLANE)),
            pltpu.SemaphoreType.DMA((COL - 1, PLANE)),
            pltpu.SemaphoreType.DMA((PLANE - 1,)),
            pltpu.SemaphoreType.DMA((PLANE - 1,)),
            pltpu.SemaphoreType.DMA((PLANE - 1,)),
            pltpu.SemaphoreType.DMA((PLANE - 1,)),
        ],
        compiler_params=pltpu.CompilerParams(
            collective_id=0,
            vmem_limit_bytes=100 * 1024 * 1024,
        ),
    )(x, w_mat, scale_x, scale_w, meta)


# device time: 92964 ns/iter; 1.0343x vs baseline; 1.0343x over previous
import jax
import jax.numpy as jnp
from jax import lax
from jax.experimental import pallas as pl
from jax.experimental.pallas import tpu as pltpu

N_DEV = 32
PLANE = 8
COL = 4

CYC = [0, 1, 2, 5, 6, 7, 4, 3]
POS = [0, 1, 2, 7, 6, 3, 4, 5]

WA = 640
WB = 384
A_LO, A_HI = 0, WA
B_LO, B_HI = 2 * WA, 2 * WA + WB


def kernel(x, w_mat, scale_x, scale_w):
    m_glob, k_loc = x.shape
    _, n = w_mat.shape
    m_blk = m_glob // N_DEV

    d = lax.axis_index("i")
    z = d // PLANE
    idx = d % PLANE
    cyc = jnp.array(CYC, jnp.int32)
    q = jnp.array(POS, jnp.int32)[idx]
    succ = z * PLANE + cyc[(q + 1) % PLANE]
    pred = z * PLANE + cyc[(q + 7) % PLANE]
    zsucc = ((z + 1) % COL) * PLANE + idx
    zpred = ((z + 3) % COL) * PLANE + idx
    meta = jnp.concatenate(
        [jnp.stack([z, q, succ, pred, zsucc, zpred]).astype(jnp.int32), cyc]
    )

    def body(x_ref, w_ref, sx_ref, sw_ref, meta_ref, out_ref,
             ga_p, ga_m, gb_p, gb_m, ha_p, ha_m, hb_p, hb_m,
             a1sp, a1rp, a1sm, a1rm, a2sp, a2rp, a2sm, a2rm,
             b1sp, b1rp, b1sm, b1rm, b2sp, b2rp, b2sm, b2rm):
        z = meta_ref[0]
        q = meta_ref[1]
        succ = meta_ref[2]
        pred = meta_ref[3]
        zsucc = meta_ref[4]
        zpred = meta_ref[5]

        def cyc_at_(p):
            return meta_ref[6 + p]

        barrier = pltpu.get_barrier_semaphore()
        for k in range(1, PLANE):
            tgt = z * PLANE + cyc_at_(lax.rem(q + k, PLANE))
            pl.semaphore_signal(barrier, inc=1, device_id=(tgt,),
                                device_id_type=pl.DeviceIdType.MESH)
        for nbr in (zsucc, zpred):
            pl.semaphore_signal(barrier, inc=1, device_id=(nbr,),
                                device_id_type=pl.DeviceIdType.MESH)
        pl.semaphore_wait(barrier, PLANE - 1 + 2)

        def pmat(c, off, wdt):
            xb = x_ref[pl.ds(c * m_blk, m_blk), :]
            return lax.dot_general(
                xb, w_ref[:, off:off + wdt], (((1,), (0,)), ((), ())),
                preferred_element_type=jnp.float32)

        def cyc_at(p):
            return meta_ref[6 + p]

        def rdma(buf, sends, recvs, slot_s, slot_d, lane, tgt):
            if lane is None:
                return pltpu.make_async_remote_copy(
                    src_ref=buf.at[slot_s], dst_ref=buf.at[slot_d],
                    send_sem=sends.at[slot_s], recv_sem=recvs.at[slot_s],
                    device_id=(tgt,), device_id_type=pl.DeviceIdType.MESH)
            return pltpu.make_async_remote_copy(
                src_ref=buf.at[slot_s, lane], dst_ref=buf.at[slot_d, lane],
                send_sem=sends.at[slot_s, lane], recv_sem=recvs.at[slot_s, lane],
                device_id=(tgt,), device_id_type=pl.DeviceIdType.MESH)

        bf = jnp.bfloat16
        f32 = jnp.float32

        iap = cyc_at(lax.rem(q + 7, PLANE))
        iam = cyc_at(lax.rem(q + 1, PLANE))
        for zc in range(COL):
            ga_p[0, zc] = pmat(zc * PLANE + iap, A_LO, WA).astype(bf)
            rdma(ga_p, a1sp, a1rp, 0, 1, zc, succ).start()
            ga_m[0, zc] = pmat(zc * PLANE + iam, A_HI, WA).astype(bf)
            rdma(ga_m, a1sm, a1rm, 0, 1, zc, pred).start()

        zbp = lax.rem(z + 3, COL)
        zbm = lax.rem(z + 1, COL)
        for i in range(PLANE):
            gb_p[0, i] = pmat(zbp * PLANE + i, B_LO, WB).astype(bf)
            rdma(gb_p, b1sp, b1rp, 0, 1, i, zsucc).start()
            gb_m[0, i] = pmat(zbm * PLANE + i, B_HI, WB).astype(bf)
            rdma(gb_m, b1sm, b1rm, 0, 1, i, zpred).start()

        def a1_hop(h, fire_a2=False):
            ip = cyc_at(lax.rem(q + 15 - h, PLANE))
            im = cyc_at(lax.rem(q + h + 1, PLANE))
            nxt = (pmat(ip, A_LO, WA), pmat(im, A_HI, WA))
            for zc in range(COL):
                pp, pm = nxt
                if zc < COL - 1:
                    nxt = (pmat((zc + 1) * PLANE + ip, A_LO, WA),
                           pmat((zc + 1) * PLANE + im, A_HI, WA))
                rdma(ga_p, a1sp, a1rp, h - 1, h, zc, succ).wait_recv()
                ga_p[h, zc] = (ga_p[h, zc].astype(f32) + pp).astype(bf)
                if h < PLANE - 1:
                    rdma(ga_p, a1sp, a1rp, h, h + 1, zc, succ).start()
                rdma(ga_m, a1sm, a1rm, h - 1, h, zc, pred).wait_recv()
                ga_m[h, zc] = (ga_m[h, zc].astype(f32) + pm).astype(bf)
                if h < PLANE - 1:
                    rdma(ga_m, a1sm, a1rm, h, h + 1, zc, pred).start()
                if fire_a2:
                    @pl.when(zc == lax.rem(z + 3, COL))
                    def _():
                        ha_p[0] = ga_p[PLANE - 1, zc]
                        rdma(ha_p, a2sp, a2rp, 0, 1, None, zsucc).start()

                    @pl.when(zc == lax.rem(z + 1, COL))
                    def _():
                        ha_m[0] = ga_m[PLANE - 1, zc]
                        rdma(ha_m, a2sm, a2rm, 0, 1, None, zpred).start()

        def b1_hop(t, fire_b2=False):
            zp = lax.rem(z + 7 - t, COL)
            zm = lax.rem(z + t + 1, COL)
            nxt = (pmat(zp * PLANE, B_LO, WB), pmat(zm * PLANE, B_HI, WB))
            for i in range(PLANE):
                pp, pm = nxt
                if i < PLANE - 1:
                    nxt = (pmat(zp * PLANE + i + 1, B_LO, WB),
                           pmat(zm * PLANE + i + 1, B_HI, WB))
                rdma(gb_p, b1sp, b1rp, t - 1, t, i, zsucc).wait_recv()
                gb_p[t, i] = (gb_p[t, i].astype(f32) + pp).astype(bf)
                if t < COL - 1:
                    rdma(gb_p, b1sp, b1rp, t, t + 1, i, zsucc).start()
                rdma(gb_m, b1sm, b1rm, t - 1, t, i, zpred).wait_recv()
                gb_m[t, i] = (gb_m[t, i].astype(f32) + pm).astype(bf)
                if t < COL - 1:
                    rdma(gb_m, b1sm, b1rm, t, t + 1, i, zpred).start()
        a1_hop(1)
        a1_hop(2)
        b1_hop(1)
        a1_hop(3)
        a1_hop(4)
        b1_hop(2)
        a1_hop(5)
        a1_hop(6)
        b1_hop(3)

        for k in range(1, PLANE):
            lane = cyc_at(lax.rem(q + k, PLANE))
            tgt = z * PLANE + lane
            pltpu.make_async_remote_copy(
                src_ref=gb_p.at[COL - 1, lane], dst_ref=hb_p.at[k - 1],
                send_sem=b2sp.at[k - 1], recv_sem=b2rp.at[k - 1],
                device_id=(tgt,), device_id_type=pl.DeviceIdType.MESH,
            ).start()
            pltpu.make_async_remote_copy(
                src_ref=gb_m.at[COL - 1, lane], dst_ref=hb_m.at[k - 1],
                send_sem=b2sm.at[k - 1], recv_sem=b2rm.at[k - 1],
                device_id=(tgt,), device_id_type=pl.DeviceIdType.MESH,
            ).start()

        a1_hop(7, fire_a2=True)

        s = sx_ref[0] * sw_ref[0]

        def a2_hop(t):
            addp = ga_p[PLANE - 1, lax.rem(z + 7 - t, COL)].astype(f32)
            addm = ga_m[PLANE - 1, lax.rem(z + t + 1, COL)].astype(f32)
            rdma(ha_p, a2sp, a2rp, t - 1, t, None, zsucc).wait_recv()
            accp = ha_p[t].astype(f32) + addp
            if t < COL - 1:
                ha_p[t] = accp.astype(bf)
                rdma(ha_p, a2sp, a2rp, t, t + 1, None, zsucc).start()
            else:
                out_ref[:, A_LO:A_LO + WA] = jnp.maximum(accp * s, 0.0)
            rdma(ha_m, a2sm, a2rm, t - 1, t, None, zpred).wait_recv()
            accm = ha_m[t].astype(f32) + addm
            if t < COL - 1:
                ha_m[t] = accm.astype(bf)
                rdma(ha_m, a2sm, a2rm, t, t + 1, None, zpred).start()
            else:
                out_ref[:, A_HI:A_HI + WA] = jnp.maximum(accm * s, 0.0)

        def b2_recv(k):
            r = pltpu.make_async_remote_copy(
                src_ref=gb_p.at[COL - 1, 0], dst_ref=hb_p.at[k - 1],
                send_sem=b2sp.at[k - 1], recv_sem=b2rp.at[k - 1],
                device_id=(succ,), device_id_type=pl.DeviceIdType.MESH,
            )
            r.wait_recv()
            m = pltpu.make_async_remote_copy(
                src_ref=gb_m.at[COL - 1, 0], dst_ref=hb_m.at[k - 1],
                send_sem=b2sm.at[k - 1], recv_sem=b2rm.at[k - 1],
                device_id=(succ,), device_id_type=pl.DeviceIdType.MESH,
            )
            m.wait_recv()

        my_idx = cyc_at(q)
        accp = gb_p[COL - 1, my_idx].astype(f32)
        accm = gb_m[COL - 1, my_idx].astype(f32)
        a2_hop(1)
        for k in range(1, PLANE):
            b2_recv(k)
            accp = accp + hb_p[k - 1].astype(f32)
            accm = accm + hb_m[k - 1].astype(f32)
            if k == 2:
                a2_hop(2)
            if k == 4:
                a2_hop(3)
        out_ref[:, B_LO:B_LO + WB] = jnp.maximum(accp * s, 0.0)
        out_ref[:, B_HI:B_HI + WB] = jnp.maximum(accm * s, 0.0)

        for h in range(1, PLANE):
            for zc in range(COL):
                rdma(ga_p, a1sp, a1rp, h - 1, h, zc, succ).wait_send()
                rdma(ga_m, a1sm, a1rm, h - 1, h, zc, pred).wait_send()
        for t in range(1, COL):
            for i in range(PLANE):
                rdma(gb_p, b1sp, b1rp, t - 1, t, i, zsucc).wait_send()
                rdma(gb_m, b1sm, b1rm, t - 1, t, i, zpred).wait_send()
        for k in range(1, PLANE):
            pltpu.make_async_remote_copy(
                src_ref=gb_p.at[COL - 1, 0], dst_ref=hb_p.at[k - 1],
                send_sem=b2sp.at[k - 1], recv_sem=b2rp.at[k - 1],
                device_id=(succ,), device_id_type=pl.DeviceIdType.MESH,
            ).wait_send()
            pltpu.make_async_remote_copy(
                src_ref=gb_m.at[COL - 1, 0], dst_ref=hb_m.at[k - 1],
                send_sem=b2sm.at[k - 1], recv_sem=b2rm.at[k - 1],
                device_id=(succ,), device_id_type=pl.DeviceIdType.MESH,
            ).wait_send()
        for t in range(1, COL):
            rdma(ha_p, a2sp, a2rp, t - 1, t, None, zsucc).wait_send()
            rdma(ha_m, a2sm, a2rm, t - 1, t, None, zpred).wait_send()

    return pl.pallas_call(
        body,
        out_shape=jax.ShapeDtypeStruct((m_blk, n), jnp.float32),
        in_specs=[
            pl.BlockSpec(memory_space=pltpu.VMEM),
            pl.BlockSpec(memory_space=pltpu.VMEM),
            pl.BlockSpec(memory_space=pltpu.SMEM),
            pl.BlockSpec(memory_space=pltpu.SMEM),
            pl.BlockSpec(memory_space=pltpu.SMEM),
        ],
        out_specs=pl.BlockSpec(memory_space=pltpu.VMEM),
        scratch_shapes=[
            pltpu.VMEM((PLANE, COL, m_blk, WA), jnp.bfloat16),
            pltpu.VMEM((PLANE, COL, m_blk, WA), jnp.bfloat16),
            pltpu.VMEM((COL, PLANE, m_blk, WB), jnp.bfloat16),
            pltpu.VMEM((COL, PLANE, m_blk, WB), jnp.bfloat16),
            pltpu.VMEM((COL, m_blk, WA), jnp.bfloat16),
            pltpu.VMEM((COL, m_blk, WA), jnp.bfloat16),
            pltpu.VMEM((PLANE, m_blk, WB), jnp.bfloat16),
            pltpu.VMEM((PLANE, m_blk, WB), jnp.bfloat16),
            pltpu.SemaphoreType.DMA((PLANE - 1, COL)),
            pltpu.SemaphoreType.DMA((PLANE - 1, COL)),
            pltpu.SemaphoreType.DMA((PLANE - 1, COL)),
            pltpu.SemaphoreType.DMA((PLANE - 1, COL)),
            pltpu.SemaphoreType.DMA((COL - 1,)),
            pltpu.SemaphoreType.DMA((COL - 1,)),
            pltpu.SemaphoreType.DMA((COL - 1,)),
            pltpu.SemaphoreType.DMA((COL - 1,)),
            pltpu.SemaphoreType.DMA((COL - 1, PLANE)),
            pltpu.SemaphoreType.DMA((COL - 1, PLANE)),
            pltpu.SemaphoreType.DMA((COL - 1, PLANE)),
            pltpu.SemaphoreType.DMA((COL - 1, PLANE)),
            pltpu.SemaphoreType.DMA((PLANE - 1,)),
            pltpu.SemaphoreType.DMA((PLANE - 1,)),
            pltpu.SemaphoreType.DMA((PLANE - 1,)),
            pltpu.SemaphoreType.DMA((PLANE - 1,)),
        ],
        compiler_params=pltpu.CompilerParams(
            collective_id=0,
            vmem_limit_bytes=100 * 1024 * 1024,
        ),
    )(x, w_mat, scale_x, scale_w, meta)
